# baseline (device time: 150525 ns/iter reference)
import jax
import jax.numpy as jnp
from jax import lax
from jax.experimental import pallas as pl
from jax.experimental.pallas import tpu as pltpu

N_DEV = 4
B = 8
B_PER = 2
SQ = 512
D = 1024
H_PER = 8
DH = 128
HD2 = DH // 2
SCALE = 0.08838834764831843
LOG2E = 1.4426950408889634

import numpy as _np

def _wqkv_cols():
    cols = []
    for h in range(H_PER):
        base = h * DH
        qe = [base + 2 * k for k in range(HD2)]
        qo = [base + 2 * k + 1 for k in range(HD2)]
        cols += qe + qo
        cols += [H_PER * DH + c for c in qe + qo]
        cols += [2 * H_PER * DH + base + j for j in range(DH)]
    return _np.asarray(cols, dtype=_np.int32)

_WQKV_COLS = _wqkv_cols()

_sem_signal = getattr(pl, "semaphore_signal", None) or pltpu.semaphore_signal
_sem_wait = getattr(pl, "semaphore_wait", None) or pltpu.semaphore_wait
_DevIdType = getattr(pl, "DeviceIdType", None) or pltpu.DeviceIdType
_CompilerParams = getattr(pltpu, "CompilerParams", None) or pltpu.TPUCompilerParams


def _body(
    cosq_ref,
    sinq_ref,
    cos_ref,
    sin_ref,
    x_ref,
    wqkv_ref,
    wo_ref,
    out_ref,
    xcr,
    xcl,
    y0,
    s_r,
    s_l,
    r_r,
    r_l,
    ctx_buf,
    agr_send,
    agr_recv,
    agl_send,
    agl_recv,
    rsr_send,
    rsr_recv,
    rsl_send,
    rsl_recv,
):
    my = lax.axis_index("i")
    left = lax.rem(my + N_DEV - 1, N_DEV)
    right = lax.rem(my + 1, N_DEV)

    barrier_sem = pltpu.get_barrier_semaphore()
    for nbr in (left, right):
        _sem_signal(
            barrier_sem, inc=1, device_id=(nbr,), device_id_type=_DevIdType.MESH
        )
    _sem_wait(barrier_sem, 2)

    cosq = cosq_ref[...]
    sinq = sinq_ref[...]
    cos = cos_ref[...]
    sin = sin_ref[...]

    def compute_unit(x_slot, dst_ref):
        x = x_slot[...]

        def head_body(h, _):
            qkv = jnp.dot(
                x,
                wqkv_ref[:, pl.ds(h * (3 * DH), 3 * DH)],
                preferred_element_type=jnp.float32,
            )
            q, k, v = qkv[:, :DH], qkv[:, DH : 2 * DH], qkv[:, 2 * DH :]
            q1, q2 = q[:, :HD2], q[:, HD2:]
            k1, k2 = k[:, :HD2], k[:, HD2:]
            qr = jnp.concatenate(
                [q1 * cosq - q2 * sinq, q2 * cosq + q1 * sinq], axis=1
            ).astype(jnp.bfloat16)
            kr = jnp.concatenate(
                [k1 * cos - k2 * sin, k2 * cos + k1 * sin], axis=1
            ).astype(jnp.bfloat16)
            s = lax.dot_general(
                qr,
                kr,
                (((1,), (1,)), ((), ())),
                preferred_element_type=jnp.float32,
            )
            e = jnp.exp2(s)
            r = 1.0 / jnp.sum(e, axis=1, keepdims=True)
            ctx = (
                jnp.dot(
                    e.astype(jnp.bfloat16),
                    v.astype(jnp.bfloat16),
                    preferred_element_type=jnp.float32,
                )
                * r
            )
            ctx_buf[:, pl.ds(h * DH, DH)] = ctx.astype(jnp.bfloat16)
            return 0

        lax.fori_loop(0, H_PER, head_body, 0, unroll=4)
        dst_ref[...] = jnp.dot(
            ctx_buf[...], wo_ref[...], preferred_element_type=jnp.float32
        ).astype(jnp.bfloat16)

    def ring_rdma(src, dst, ssem, rsem, dev):
        return pltpu.make_async_remote_copy(
            src_ref=src,
            dst_ref=dst,
            send_sem=ssem,
            recv_sem=rsem,
            device_id=(dev,),
            device_id_type=_DevIdType.MESH,
        )

    def agr(h):
        return ring_rdma(
            x_ref.at[0] if h == 0 else xcr.at[h - 1],
            xcr.at[h],
            agr_send.at[h],
            agr_recv.at[h],
            right,
        )

    def agl(h):
        return ring_rdma(
            x_ref.at[1] if h == 0 else xcl.at[h - 1],
            xcl.at[h],
            agl_send.at[h],
            agl_recv.at[h],
            left,
        )

    def rsr(h):
        return ring_rdma(
            s_r.at[h % 2],
            r_r.at[h],
            rsr_send.at[h],
            rsr_recv.at[h],
            right,
        )

    def rsl(h):
        return ring_rdma(
            s_l.at[h % 2],
            r_l.at[h],
            rsl_send.at[h],
            rsl_recv.at[h],
            left,
        )

    agr0 = agr(0)
    agr0.start()
    agl0 = agl(0)
    agl0.start()
    compute_unit(x_ref.at[0], y0.at[0])
    compute_unit(x_ref.at[1], y0.at[1])
    agr0.wait()
    agr1 = agr(1)
    agr1.start()
    agl0.wait()
    agl1 = agl(1)
    agl1.start()

    compute_unit(xcr.at[0], s_r.at[0])
    rsr0 = rsr(0)
    rsr0.start()
    compute_unit(xcl.at[0], s_l.at[0])
    rsl0 = rsl(0)
    rsl0.start()
    agr1.wait()
    agr2 = agr(2)
    agr2.start()
    agl1.wait()
    agl2 = agl(2)
    agl2.start()

    compute_unit(xcr.at[1], s_r.at[1])
    rsr0.wait()
    s_r[1] += r_r[0]
    rsr1 = rsr(1)
    rsr1.start()
    compute_unit(xcl.at[1], s_l.at[1])
    rsl0.wait()
    s_l[1] += r_l[0]
    rsl1 = rsl(1)
    rsl1.start()
    agr2.wait()
    agl2.wait()

    compute_unit(xcr.at[2], s_r.at[0])
    rsr1.wait()
    s_r[0] += r_r[1]
    rsr2 = rsr(2)
    rsr2.start()
    compute_unit(xcl.at[2], s_l.at[0])
    rsl1.wait()
    s_l[0] += r_l[1]
    rsl2 = rsl(2)
    rsl2.start()

    rsr2.wait()
    out_ref[0] = (r_r[2] + y0[0]).astype(jnp.float32)
    rsl2.wait()
    out_ref[1] = (r_l[2] + y0[1]).astype(jnp.float32)


def kernel(x, Wq, Wk, Wv, Wo):
    inv = 1.0 / (10000.0 ** (jnp.arange(0, DH, 2, dtype=jnp.float32) / DH))
    pos = jnp.arange(SQ, dtype=jnp.float32)[:, None] * inv[None, :]
    cos = jnp.cos(pos)
    sin = jnp.sin(pos)

    wqkv_r = jnp.concatenate(
        [Wq.astype(jnp.bfloat16), Wk.astype(jnp.bfloat16), Wv.astype(jnp.bfloat16)],
        axis=1,
    )[:, _WQKV_COLS]
    wo_r = Wo.astype(jnp.bfloat16)
    x16 = x.astype(jnp.bfloat16)

    half = (SQ, D)
    return pl.pallas_call(
        _body,
        out_shape=jax.ShapeDtypeStruct((B_PER, SQ, D), jnp.float32),
        in_specs=[pl.BlockSpec(memory_space=pltpu.VMEM)] * 7,
        out_specs=pl.BlockSpec(memory_space=pltpu.VMEM),
        scratch_shapes=[
            pltpu.VMEM((3, *half), jnp.bfloat16),
            pltpu.VMEM((3, *half), jnp.bfloat16),
            pltpu.VMEM((2, *half), jnp.bfloat16),
            pltpu.VMEM((2, *half), jnp.bfloat16),
            pltpu.VMEM((2, *half), jnp.bfloat16),
            pltpu.VMEM((3, *half), jnp.bfloat16),
            pltpu.VMEM((3, *half), jnp.bfloat16),
            pltpu.VMEM(half, jnp.bfloat16),
            pltpu.SemaphoreType.DMA((3,)),
            pltpu.SemaphoreType.DMA((3,)),
            pltpu.SemaphoreType.DMA((3,)),
            pltpu.SemaphoreType.DMA((3,)),
            pltpu.SemaphoreType.DMA((3,)),
            pltpu.SemaphoreType.DMA((3,)),
            pltpu.SemaphoreType.DMA((3,)),
            pltpu.SemaphoreType.DMA((3,)),
        ],
        compiler_params=_CompilerParams(
            collective_id=0,
            vmem_limit_bytes=100 * 1024 * 1024,
            skip_device_barrier=True,
        ),
    )(cos * (SCALE * LOG2E), sin * (SCALE * LOG2E), cos, sin, x16, wqkv_r, wo_r)


# device time: 144424 ns/iter; 1.0422x vs baseline; 1.0422x over previous
import jax
import jax.numpy as jnp
from jax import lax
from jax.experimental import pallas as pl
from jax.experimental.pallas import tpu as pltpu

N_DEV = 4
B = 8
B_PER = 2
SQ = 512
D = 1024
H_PER = 8
DH = 128
HD2 = DH // 2
SCALE = 0.08838834764831843
LOG2E = 1.4426950408889634

_sem_signal = getattr(pl, "semaphore_signal", None) or pltpu.semaphore_signal
_sem_wait = getattr(pl, "semaphore_wait", None) or pltpu.semaphore_wait
_DevIdType = getattr(pl, "DeviceIdType", None) or pltpu.DeviceIdType
_CompilerParams = getattr(pltpu, "CompilerParams", None) or pltpu.TPUCompilerParams


def _body(
    cosq_ref,
    sinq_ref,
    cos_ref,
    sin_ref,
    x_ref,
    wqkv_ref,
    wo_ref,
    out_ref,
    xcr,
    xcl,
    y0,
    s_r,
    s_l,
    r_r,
    r_l,
    ctx_buf,
    agr_send,
    agr_recv,
    agl_send,
    agl_recv,
    rsr_send,
    rsr_recv,
    rsl_send,
    rsl_recv,
):
    my = lax.axis_index("i")
    left = lax.rem(my + N_DEV - 1, N_DEV)
    right = lax.rem(my + 1, N_DEV)

    barrier_sem = pltpu.get_barrier_semaphore()
    for nbr in (left, right):
        _sem_signal(
            barrier_sem, inc=1, device_id=(nbr,), device_id_type=_DevIdType.MESH
        )
    _sem_wait(barrier_sem, 2)

    cosq = cosq_ref[...]
    sinq = sinq_ref[...]
    cos = cos_ref[...]
    sin = sin_ref[...]

    def compute_unit(x_slot, dst_ref):
        x = x_slot[...]

        def head_body(h, _):
            qkv = jnp.dot(
                x, wqkv_ref[h], preferred_element_type=jnp.float32
            )
            q, k, v = qkv[:, :DH], qkv[:, DH : 2 * DH], qkv[:, 2 * DH :]
            q1, q2 = q[:, :HD2], q[:, HD2:]
            k1, k2 = k[:, :HD2], k[:, HD2:]
            qr = jnp.concatenate(
                [q1 * cosq - q2 * sinq, q2 * cosq + q1 * sinq], axis=1
            ).astype(jnp.bfloat16)
            kr = jnp.concatenate(
                [k1 * cos - k2 * sin, k2 * cos + k1 * sin], axis=1
            ).astype(jnp.bfloat16)
            s = lax.dot_general(
                qr,
                kr,
                (((1,), (1,)), ((), ())),
                preferred_element_type=jnp.float32,
            )
            e = jnp.exp2(s)
            r = 1.0 / jnp.sum(e, axis=1, keepdims=True)
            ctx = (
                jnp.dot(
                    e.astype(jnp.bfloat16),
                    v.astype(jnp.bfloat16),
                    preferred_element_type=jnp.float32,
                )
                * r
            )
            ctx_buf[:, pl.ds(h * DH, DH)] = ctx.astype(jnp.bfloat16)
            return 0

        lax.fori_loop(0, H_PER, head_body, 0, unroll=4)
        dst_ref[...] = jnp.dot(
            ctx_buf[...], wo_ref[...], preferred_element_type=jnp.float32
        ).astype(jnp.bfloat16)

    def ring_rdma(src, dst, ssem, rsem, dev):
        return pltpu.make_async_remote_copy(
            src_ref=src,
            dst_ref=dst,
            send_sem=ssem,
            recv_sem=rsem,
            device_id=(dev,),
            device_id_type=_DevIdType.MESH,
        )

    def agr(h):
        return ring_rdma(
            x_ref.at[0] if h == 0 else xcr.at[h - 1],
            xcr.at[h],
            agr_send.at[h],
            agr_recv.at[h],
            right,
        )

    def agl(h):
        return ring_rdma(
            x_ref.at[1] if h == 0 else xcl.at[h - 1],
            xcl.at[h],
            agl_send.at[h],
            agl_recv.at[h],
            left,
        )

    def rsr(h):
        return ring_rdma(
            s_r.at[h % 2],
            r_r.at[h],
            rsr_send.at[h],
            rsr_recv.at[h],
            right,
        )

    def rsl(h):
        return ring_rdma(
            s_l.at[h % 2],
            r_l.at[h],
            rsl_send.at[h],
            rsl_recv.at[h],
            left,
        )

    agr0 = agr(0)
    agr0.start()
    agl0 = agl(0)
    agl0.start()
    compute_unit(x_ref.at[0], y0.at[0])
    agr0.wait()
    agr1 = agr(1)
    agr1.start()
    compute_unit(x_ref.at[1], y0.at[1])
    agl0.wait()
    agl1 = agl(1)
    agl1.start()

    compute_unit(xcr.at[0], s_r.at[0])
    rsr0 = rsr(0)
    rsr0.start()
    agr1.wait()
    agr2 = agr(2)
    agr2.start()
    compute_unit(xcl.at[0], s_l.at[0])
    rsl0 = rsl(0)
    rsl0.start()
    agl1.wait()
    agl2 = agl(2)
    agl2.start()

    compute_unit(xcr.at[1], s_r.at[1])
    rsr0.wait()
    s_r[1] += r_r[0]
    rsr1 = rsr(1)
    rsr1.start()
    agr2.wait()
    compute_unit(xcl.at[1], s_l.at[1])
    rsl0.wait()
    s_l[1] += r_l[0]
    rsl1 = rsl(1)
    rsl1.start()
    agl2.wait()

    compute_unit(xcr.at[2], s_r.at[0])
    rsr1.wait()
    s_r[0] += r_r[1]
    rsr2 = rsr(2)
    rsr2.start()
    compute_unit(xcl.at[2], s_l.at[0])
    rsl1.wait()
    s_l[0] += r_l[1]
    rsl2 = rsl(2)
    rsl2.start()

    rsr2.wait()
    out_ref[0] = (r_r[2] + y0[0]).astype(jnp.float32)
    rsl2.wait()
    out_ref[1] = (r_l[2] + y0[1]).astype(jnp.float32)


def kernel(x, Wq, Wk, Wv, Wo):
    inv = 1.0 / (10000.0 ** (jnp.arange(0, DH, 2, dtype=jnp.float32) / DH))
    pos = jnp.arange(SQ, dtype=jnp.float32)[:, None] * inv[None, :]
    cos = jnp.cos(pos)
    sin = jnp.sin(pos)

    def perm_qk(w):
        return w.reshape(D, H_PER, HD2, 2).transpose(1, 0, 3, 2).reshape(
            H_PER, D, DH
        )

    wqkv_r = jnp.concatenate(
        [
            perm_qk(Wq.astype(jnp.bfloat16)),
            perm_qk(Wk.astype(jnp.bfloat16)),
            Wv.astype(jnp.bfloat16).reshape(D, H_PER, DH).transpose(1, 0, 2),
        ],
        axis=2,
    )
    wo_r = Wo.astype(jnp.bfloat16)
    x16 = x.astype(jnp.bfloat16)

    half = (SQ, D)
    return pl.pallas_call(
        _body,
        out_shape=jax.ShapeDtypeStruct((B_PER, SQ, D), jnp.float32),
        in_specs=[pl.BlockSpec(memory_space=pltpu.VMEM)] * 7,
        out_specs=pl.BlockSpec(memory_space=pltpu.VMEM),
        scratch_shapes=[
            pltpu.VMEM((3, *half), jnp.bfloat16),
            pltpu.VMEM((3, *half), jnp.bfloat16),
            pltpu.VMEM((2, *half), jnp.bfloat16),
            pltpu.VMEM((2, *half), jnp.bfloat16),
            pltpu.VMEM((2, *half), jnp.bfloat16),
            pltpu.VMEM((3, *half), jnp.bfloat16),
            pltpu.VMEM((3, *half), jnp.bfloat16),
            pltpu.VMEM(half, jnp.bfloat16),
            pltpu.SemaphoreType.DMA((3,)),
            pltpu.SemaphoreType.DMA((3,)),
            pltpu.SemaphoreType.DMA((3,)),
            pltpu.SemaphoreType.DMA((3,)),
            pltpu.SemaphoreType.DMA((3,)),
            pltpu.SemaphoreType.DMA((3,)),
            pltpu.SemaphoreType.DMA((3,)),
            pltpu.SemaphoreType.DMA((3,)),
        ],
        compiler_params=_CompilerParams(
            collective_id=0,
            vmem_limit_bytes=100 * 1024 * 1024,
            skip_device_barrier=True,
        ),
    )(cos * (SCALE * LOG2E), sin * (SCALE * LOG2E), cos, sin, x16, wqkv_r, wo_r)


# device time: 141384 ns/iter; 1.0647x vs baseline; 1.0215x over previous
import jax
import jax.numpy as jnp
from jax import lax
from jax.experimental import pallas as pl
from jax.experimental.pallas import tpu as pltpu

N_DEV = 4
B = 8
B_PER = 2
SQ = 512
D = 1024
H_PER = 8
DH = 128
HD2 = DH // 2
SCALE = 0.08838834764831843
LOG2E = 1.4426950408889634

_sem_signal = getattr(pl, "semaphore_signal", None) or pltpu.semaphore_signal
_sem_wait = getattr(pl, "semaphore_wait", None) or pltpu.semaphore_wait
_DevIdType = getattr(pl, "DeviceIdType", None) or pltpu.DeviceIdType
_CompilerParams = getattr(pltpu, "CompilerParams", None) or pltpu.TPUCompilerParams


def _body(
    cosq_ref,
    sinq_ref,
    cos_ref,
    sin_ref,
    x_ref,
    wqkv_ref,
    wo_ref,
    out_ref,
    xcr,
    xcl,
    y0,
    s_r,
    s_l,
    r_r,
    r_l,
    ctx_buf,
    agr_send,
    agr_recv,
    agl_send,
    agl_recv,
    rsr_send,
    rsr_recv,
    rsl_send,
    rsl_recv,
):
    my = lax.axis_index("i")
    left = lax.rem(my + N_DEV - 1, N_DEV)
    right = lax.rem(my + 1, N_DEV)

    barrier_sem = pltpu.get_barrier_semaphore()
    for nbr in (left, right):
        _sem_signal(
            barrier_sem, inc=1, device_id=(nbr,), device_id_type=_DevIdType.MESH
        )
    _sem_wait(barrier_sem, 2)

    cosq = cosq_ref[...]
    sinq = sinq_ref[...]
    cos = cos_ref[...]
    sin = sin_ref[...]

    def compute_unit(x_slot, dst_ref):
        x = x_slot[...]

        def head_body(h, _):
            qkv = jnp.dot(
                x, wqkv_ref[h], preferred_element_type=jnp.float32
            )
            q, k, v = qkv[:, :DH], qkv[:, DH : 2 * DH], qkv[:, 2 * DH :]
            q1, q2 = q[:, :HD2], q[:, HD2:]
            k1, k2 = k[:, :HD2], k[:, HD2:]
            qr = jnp.concatenate(
                [q1 * cosq - q2 * sinq, q2 * cosq + q1 * sinq], axis=1
            ).astype(jnp.bfloat16)
            kr = jnp.concatenate(
                [k1 * cos - k2 * sin, k2 * cos + k1 * sin], axis=1
            ).astype(jnp.bfloat16)
            s = lax.dot_general(
                qr,
                kr,
                (((1,), (1,)), ((), ())),
                preferred_element_type=jnp.float32,
            )
            e = jnp.exp2(s)
            r = 1.0 / jnp.sum(e, axis=1, keepdims=True)
            ctx = (
                jnp.dot(
                    e.astype(jnp.bfloat16),
                    v.astype(jnp.bfloat16),
                    preferred_element_type=jnp.float32,
                )
                * r
            )
            ctx_buf[:, pl.ds(h * DH, DH)] = ctx.astype(jnp.bfloat16)
            return 0

        lax.fori_loop(0, H_PER, head_body, 0, unroll=4)
        dst_ref[...] = jnp.dot(
            ctx_buf[...], wo_ref[...], preferred_element_type=jnp.float32
        ).astype(jnp.bfloat16)

    def ring_rdma(src, dst, ssem, rsem, dev):
        return pltpu.make_async_remote_copy(
            src_ref=src,
            dst_ref=dst,
            send_sem=ssem,
            recv_sem=rsem,
            device_id=(dev,),
            device_id_type=_DevIdType.MESH,
        )

    def agr(h):
        return ring_rdma(
            x_ref.at[0] if h == 0 else xcr.at[h - 1],
            xcr.at[h],
            agr_send.at[h],
            agr_recv.at[h],
            right,
        )

    def agl(h):
        return ring_rdma(
            x_ref.at[1] if h == 0 else xcl.at[h - 1],
            xcl.at[h],
            agl_send.at[h],
            agl_recv.at[h],
            left,
        )

    def rsr(h):
        return ring_rdma(
            s_r.at[h % 2],
            r_r.at[h],
            rsr_send.at[h],
            rsr_recv.at[h],
            right,
        )

    def rsl(h):
        return ring_rdma(
            s_l.at[h % 2],
            r_l.at[h],
            rsl_send.at[h],
            rsl_recv.at[h],
            left,
        )

    agr0 = agr(0)
    agr0.start()
    agl0 = agl(0)
    agl0.start()
    compute_unit(x_ref.at[0], y0.at[0])
    compute_unit(x_ref.at[1], y0.at[1])
    agr0.wait()
    agr1 = agr(1)
    agr1.start()
    agl0.wait()
    agl1 = agl(1)
    agl1.start()

    compute_unit(xcr.at[0], s_r.at[0])
    rsr0 = rsr(0)
    rsr0.start()
    compute_unit(xcl.at[0], s_l.at[0])
    rsl0 = rsl(0)
    rsl0.start()
    agr1.wait()
    agr2 = agr(2)
    agr2.start()
    agl1.wait()
    agl2 = agl(2)
    agl2.start()

    compute_unit(xcr.at[1], s_r.at[1])
    rsr0.wait()
    s_r[1] += r_r[0]
    rsr1 = rsr(1)
    rsr1.start()
    compute_unit(xcl.at[1], s_l.at[1])
    rsl0.wait()
    s_l[1] += r_l[0]
    rsl1 = rsl(1)
    rsl1.start()
    agr2.wait()
    agl2.wait()

    compute_unit(xcr.at[2], s_r.at[0])
    rsr1.wait()
    s_r[0] += r_r[1]
    rsr2 = rsr(2)
    rsr2.start()
    compute_unit(xcl.at[2], s_l.at[0])
    rsl1.wait()
    s_l[0] += r_l[1]
    rsl2 = rsl(2)
    rsl2.start()

    rsr2.wait()
    out_ref[0] = (r_r[2] + y0[0]).astype(jnp.float32)
    rsl2.wait()
    out_ref[1] = (r_l[2] + y0[1]).astype(jnp.float32)


def kernel(x, Wq, Wk, Wv, Wo):
    inv = 1.0 / (10000.0 ** (jnp.arange(0, DH, 2, dtype=jnp.float32) / DH))
    pos = jnp.arange(SQ, dtype=jnp.float32)[:, None] * inv[None, :]
    cos = jnp.cos(pos)
    sin = jnp.sin(pos)

    def perm_qk(w):
        return w.reshape(D, H_PER, HD2, 2).transpose(1, 0, 3, 2).reshape(
            H_PER, D, DH
        )

    wqkv_r = jnp.concatenate(
        [
            perm_qk(Wq.astype(jnp.bfloat16)),
            perm_qk(Wk.astype(jnp.bfloat16)),
            Wv.astype(jnp.bfloat16).reshape(D, H_PER, DH).transpose(1, 0, 2),
        ],
        axis=2,
    )
    wo_r = Wo.astype(jnp.bfloat16)
    x16 = x.astype(jnp.bfloat16)

    half = (SQ, D)
    return pl.pallas_call(
        _body,
        out_shape=jax.ShapeDtypeStruct((B_PER, SQ, D), jnp.float32),
        in_specs=[pl.BlockSpec(memory_space=pltpu.VMEM)] * 7,
        out_specs=pl.BlockSpec(memory_space=pltpu.VMEM),
        scratch_shapes=[
            pltpu.VMEM((3, *half), jnp.bfloat16),
            pltpu.VMEM((3, *half), jnp.bfloat16),
            pltpu.VMEM((2, *half), jnp.bfloat16),
            pltpu.VMEM((2, *half), jnp.bfloat16),
            pltpu.VMEM((2, *half), jnp.bfloat16),
            pltpu.VMEM((3, *half), jnp.bfloat16),
            pltpu.VMEM((3, *half), jnp.bfloat16),
            pltpu.VMEM(half, jnp.bfloat16),
            pltpu.SemaphoreType.DMA((3,)),
            pltpu.SemaphoreType.DMA((3,)),
            pltpu.SemaphoreType.DMA((3,)),
            pltpu.SemaphoreType.DMA((3,)),
            pltpu.SemaphoreType.DMA((3,)),
            pltpu.SemaphoreType.DMA((3,)),
            pltpu.SemaphoreType.DMA((3,)),
            pltpu.SemaphoreType.DMA((3,)),
        ],
        compiler_params=_CompilerParams(
            collective_id=0,
            vmem_limit_bytes=100 * 1024 * 1024,
            skip_device_barrier=True,
        ),
    )(cos * (SCALE * LOG2E), sin * (SCALE * LOG2E), cos, sin, x16, wqkv_r, wo_r)
